# baseline (speedup 1.0000x reference)
import jax
import jax.numpy as jnp
from jax import lax
from jax.experimental import pallas as pl
from jax.experimental.pallas import tpu as pltpu
from jax.experimental.pallas import tpu_sc as plsc

M = 500000
D = 128
HTS = 1 << 20
P0, P1, P2 = 73856093, 19349669, 83492791
NC, NS, L = 2, 16, 16
NW = NC * NS
C = 128
G_FULL = C // L
NCHUNK = M // C
TAIL_ROWS = M - NCHUNK * C
TAIL_GROUPS = TAIL_ROWS // L
ITERS = (NCHUNK + NW - 1) // NW


def _grid_floor(q):
    d = q / jnp.float32(0.1)
    t = d.astype(jnp.int32)
    return jnp.where(t.astype(jnp.float32) > d, t - 1, t)


def _hash3(qx, qy, qz):
    gx, gy, gz = _grid_floor(qx), _grid_floor(qy), _grid_floor(qz)
    s = gx * P0 + gy * P1 + gz * P2
    return s & (HTS - 1)


def _sc_body(q_hbm, tab_hbm, feat_hbm, out_hbm,
             q_v, hash_v, vox_v, sidx_v, rows_v,
             q_t, hash_t, vox_t, sidx_t, rows_t, sem):
    wid = lax.axis_index("s") * NC + lax.axis_index("c")
    lane = lax.iota(jnp.int32, L)

    def process(base, G, q_r, hash_r, vox_r, sidx_r, rows_r):
        pltpu.sync_copy(q_hbm.at[pl.ds(base * 3, G * L * 3)],
                        q_r.at[pl.ds(0, G * L * 3)])
        for g in range(G):
            off = g * L * 3
            qx = plsc.load_gather(q_r, [lane * 3 + off])
            qy = plsc.load_gather(q_r, [lane * 3 + (off + 1)])
            qz = plsc.load_gather(q_r, [lane * 3 + (off + 2)])
            hash_r[pl.ds(g * L, L)] = _hash3(qx, qy, qz) * 2
        pltpu.async_copy(tab_hbm.at[hash_r], vox_r, sem).wait()
        for g in range(G):
            vox = vox_r[pl.ds(g * L, L)]
            sidx_r[pl.ds(g * L, L)] = jnp.maximum(vox, 0)
        pltpu.async_copy(feat_hbm.at[sidx_r], rows_r, sem).wait()
        zeros = jnp.zeros((L,), jnp.float32)
        for g in range(G):
            vox = vox_r[pl.ds(g * L, L)]
            inv = vox < 0
            rows_idx = lane + g * L
            n_inv = jnp.max(inv.astype(jnp.int32), axis=0)

            @pl.when(n_inv > 0)
            def _():
                def zero_col(c, carry):
                    col = jnp.full((L,), 0, jnp.int32) + c
                    plsc.store_scatter(rows_r, [rows_idx, col], zeros,
                                       mask=inv)
                    return carry
                lax.fori_loop(jnp.int32(0), jnp.int32(D), zero_col,
                              jnp.int32(0))
        pltpu.sync_copy(rows_r, out_hbm.at[pl.ds(base, G * L)])

    def body(j, carry):
        cid = wid + j * NW

        @pl.when(cid < NCHUNK)
        def _():
            process(cid * C, G_FULL, q_v, hash_v, vox_v, sidx_v, rows_v)
        return carry

    lax.fori_loop(jnp.int32(0), jnp.int32(ITERS), body, jnp.int32(0))

    @pl.when(wid < TAIL_GROUPS)
    def _():
        process(NCHUNK * C + wid * L, 1, q_t, hash_t, vox_t, sidx_t, rows_t)


_mesh = plsc.VectorSubcoreMesh(core_axis_name="c", subcore_axis_name="s",
                               num_cores=NC, num_subcores=NS)

_sc_kernel = pl.kernel(
    _sc_body,
    out_type=jax.ShapeDtypeStruct((M, D), jnp.float32),
    mesh=_mesh,
    compiler_params=pltpu.CompilerParams(needs_layout_passes=False),
    scratch_types=[
        pltpu.VMEM((C * 3,), jnp.float32),
        pltpu.VMEM((C,), jnp.int32),
        pltpu.VMEM((C,), jnp.int32),
        pltpu.VMEM((C,), jnp.int32),
        pltpu.VMEM((C, D), jnp.float32),
        pltpu.VMEM((L * 3,), jnp.float32),
        pltpu.VMEM((L,), jnp.int32),
        pltpu.VMEM((L,), jnp.int32),
        pltpu.VMEM((L,), jnp.int32),
        pltpu.VMEM((L, D), jnp.float32),
        pltpu.SemaphoreType.DMA,
    ],
)


def kernel(query_pts, voxel_features, buffer_voxel_index):
    q_flat = query_pts.reshape(-1)
    tab = jax.lax.bitcast_convert_type(buffer_voxel_index, jnp.int32).reshape(-1)
    return _sc_kernel(q_flat, tab, voxel_features)


# speedup vs baseline: 4.3064x; 4.3064x over previous
import jax
import jax.numpy as jnp
from jax import lax
from jax.experimental import pallas as pl
from jax.experimental.pallas import tpu as pltpu
from jax.experimental.pallas import tpu_sc as plsc

M = 500000
D = 128
HTS = 1 << 20
P0, P1, P2 = 73856093, 19349669, 83492791
NC, NS, L = 2, 16, 16
NW = NC * NS
C = 128
G_FULL = C // L
NCHUNK = M // C
TAIL_ROWS = M - NCHUNK * C
TAIL_GROUPS = TAIL_ROWS // L
ITERS = (NCHUNK + NW - 1) // NW


def _grid_floor(q):
    d = q / jnp.float32(0.1)
    t = d.astype(jnp.int32)
    return jnp.where(t.astype(jnp.float32) > d, t - 1, t)


def _hash3(qx, qy, qz):
    gx, gy, gz = _grid_floor(qx), _grid_floor(qy), _grid_floor(qz)
    s = gx * P0 + gy * P1 + gz * P2
    return s & (HTS - 1)


def _sc_body(q_hbm, tab_hbm, feat_hbm, out_hbm,
             q_v, hash_v, vox_v, sidx_v, rows_v,
             q_t, hash_t, vox_t, sidx_t, rows_t, sem):
    wid = lax.axis_index("s") * NC + lax.axis_index("c")
    lane = lax.iota(jnp.int32, L)

    c0 = jnp.zeros((L,), jnp.int32)
    c1 = c0 + 1
    c2 = c0 + 2

    def process(base, G, q_r, hash_r, vox_r, sidx_r, rows_r):
        pltpu.sync_copy(q_hbm.at[pl.ds(base, G * L)], q_r)
        for g in range(G):
            rows16 = lane + g * L
            qx = plsc.load_gather(q_r, [rows16, c0])
            qy = plsc.load_gather(q_r, [rows16, c1])
            qz = plsc.load_gather(q_r, [rows16, c2])
            hash_r[pl.ds(g * L, L)] = _hash3(qx, qy, qz)
        pltpu.async_copy(tab_hbm.at[hash_r], vox_r, sem).wait()
        for g in range(G):
            vox = vox_r[pl.ds(g * L, L)]
            sidx_r[pl.ds(g * L, L)] = jnp.maximum(vox, 0)
        pltpu.async_copy(feat_hbm.at[sidx_r], rows_r, sem).wait()
        zeros = jnp.zeros((L,), jnp.float32)
        for g in range(G):
            vox = vox_r[pl.ds(g * L, L)]
            inv = vox < 0
            rows_idx = lane + g * L
            n_inv = jnp.max(inv.astype(jnp.int32), axis=0)

            @pl.when(n_inv > 0)
            def _():
                def zero_col(c, carry):
                    col = jnp.full((L,), 0, jnp.int32) + c
                    plsc.store_scatter(rows_r, [rows_idx, col], zeros,
                                       mask=inv)
                    return carry
                lax.fori_loop(jnp.int32(0), jnp.int32(D), zero_col,
                              jnp.int32(0))
        pltpu.sync_copy(rows_r, out_hbm.at[pl.ds(base, G * L)])

    def body(j, carry):
        cid = wid + j * NW

        @pl.when(cid < NCHUNK)
        def _():
            process(cid * C, G_FULL, q_v, hash_v, vox_v, sidx_v, rows_v)
        return carry

    lax.fori_loop(jnp.int32(0), jnp.int32(ITERS), body, jnp.int32(0))

    @pl.when(wid < TAIL_GROUPS)
    def _():
        process(NCHUNK * C + wid * L, 1, q_t, hash_t, vox_t, sidx_t, rows_t)


_mesh = plsc.VectorSubcoreMesh(core_axis_name="c", subcore_axis_name="s",
                               num_cores=NC, num_subcores=NS)

_sc_kernel = pl.kernel(
    _sc_body,
    out_type=jax.ShapeDtypeStruct((M, D), jnp.float32),
    mesh=_mesh,
    compiler_params=pltpu.CompilerParams(needs_layout_passes=False),
    scratch_types=[
        pltpu.VMEM((C, 3), jnp.float32),
        pltpu.VMEM((C,), jnp.int32),
        pltpu.VMEM((C,), jnp.int32),
        pltpu.VMEM((C,), jnp.int32),
        pltpu.VMEM((C, D), jnp.float32),
        pltpu.VMEM((L, 3), jnp.float32),
        pltpu.VMEM((L,), jnp.int32),
        pltpu.VMEM((L,), jnp.int32),
        pltpu.VMEM((L,), jnp.int32),
        pltpu.VMEM((L, D), jnp.float32),
        pltpu.SemaphoreType.DMA,
    ],
)


def kernel(query_pts, voxel_features, buffer_voxel_index):
    tab = buffer_voxel_index.astype(jnp.int32)
    return _sc_kernel(query_pts, tab, voxel_features)
